# Initial kernel scaffold; baseline (speedup 1.0000x reference)
import functools

import jax
import jax.numpy as jnp
from jax import lax
from jax.experimental import pallas as pl
from jax.experimental.pallas import tpu as pltpu
from jax.experimental.pallas import tpu_sc as plsc

_N_NODES = 10000
_N_EDGES = 320000
_D_IN = 128
_D_EDGE = 16
_D_OUT = 128

_NC = 2
_NS = 16
_L = 16
_NW = _NC * _NS

_K = 80
_EDGES_PER_WORKER = _N_EDGES // _NW
_CHUNKS_PER_WORKER = _EDGES_PER_WORKER // _K
_ROWS_PER_SUB = _N_NODES // _NS
_ZROWS = 125


def _sc_segment_sums(x, src, dst, edge_attr):
    mesh = plsc.VectorSubcoreMesh(core_axis_name="c", subcore_axis_name="s")

    @functools.partial(
        pl.kernel,
        out_type=[
            jax.ShapeDtypeStruct((_NC, _N_NODES, _D_IN), jnp.float32),
            jax.ShapeDtypeStruct((_NC, _N_NODES, _D_EDGE), jnp.float32),
        ],
        mesh=mesh,
        scratch_types=[
            pltpu.VMEM_SHARED((_N_NODES, _D_IN), jnp.float32),
            pltpu.VMEM_SHARED((_N_NODES, _D_EDGE), jnp.float32),
            pltpu.VMEM((_ZROWS, _D_IN), jnp.float32),
            pltpu.VMEM((_ZROWS, _D_EDGE), jnp.float32),
            pltpu.VMEM((_K,), jnp.int32),
            pltpu.VMEM((_K,), jnp.int32),
            pltpu.VMEM((_K, _D_IN), jnp.float32),
            pltpu.VMEM((_K, _D_EDGE), jnp.float32),
        ],
    )
    def k(x_hbm, src_hbm, dst_hbm, ea_hbm, outx_hbm, oute_hbm,
          acc_x, acc_e, zbuf_x, zbuf_e, src_v, dst_v, rows_v, ea_v):
        c = lax.axis_index("c")
        s = lax.axis_index("s")
        w = c * _NS + s

        @pl.loop(0, _ZROWS)
        def _(i):
            @pl.loop(0, _D_IN, step=_L)
            def _(j):
                zbuf_x[pl.ds(i, 1), pl.ds(j, _L)] = jnp.zeros((1, _L), jnp.float32)
            zbuf_e[pl.ds(i, 1), pl.ds(0, _L)] = jnp.zeros((1, _L), jnp.float32)

        r0 = s * _ROWS_PER_SUB

        @pl.loop(0, _ROWS_PER_SUB, step=_ZROWS)
        def _(r):
            pltpu.sync_copy(zbuf_x, acc_x.at[pl.ds(r0 + r, _ZROWS)])
            pltpu.sync_copy(zbuf_e, acc_e.at[pl.ds(r0 + r, _ZROWS)])

        plsc.subcore_barrier()

        base = w * _EDGES_PER_WORKER

        @pl.loop(0, _CHUNKS_PER_WORKER)
        def _(i):
            off = base + i * _K
            pltpu.sync_copy(src_hbm.at[pl.ds(off, _K)], src_v)
            pltpu.sync_copy(dst_hbm.at[pl.ds(off, _K)], dst_v)
            pltpu.sync_copy(ea_hbm.at[pl.ds(off, _K)], ea_v)
            pltpu.sync_copy(x_hbm.at[src_v], rows_v)
            pltpu.sync_copy(rows_v, acc_x.at[dst_v], add=True)
            pltpu.sync_copy(ea_v, acc_e.at[dst_v], add=True)

        plsc.subcore_barrier()

        pltpu.sync_copy(acc_x.at[pl.ds(r0, _ROWS_PER_SUB)],
                        outx_hbm.at[c].at[pl.ds(r0, _ROWS_PER_SUB)])
        pltpu.sync_copy(acc_e.at[pl.ds(r0, _ROWS_PER_SUB)],
                        oute_hbm.at[c].at[pl.ds(r0, _ROWS_PER_SUB)])

    return k(x, src, dst, edge_attr)


_BLK = 1000


def _tc_combine(px, pe, w_msg, bias):
    def body(px_ref, pe_ref, w_ref, b_ref, o_ref):
        xs = px_ref[0] + px_ref[1]
        es = pe_ref[0] + pe_ref[1]
        wx = w_ref[0:_D_IN, :]
        we = w_ref[_D_IN:_D_IN + _D_EDGE, :]
        acc = lax.dot_general(xs, wx, (((1,), (0,)), ((), ())),
                              precision=lax.Precision.HIGHEST,
                              preferred_element_type=jnp.float32)
        acc = acc + lax.dot_general(es, we, (((1,), (0,)), ((), ())),
                                    precision=lax.Precision.HIGHEST,
                                    preferred_element_type=jnp.float32)
        o_ref[...] = acc + b_ref[...]

    return pl.pallas_call(
        body,
        grid=(_N_NODES // _BLK,),
        in_specs=[
            pl.BlockSpec((_NC, _BLK, _D_IN), lambda i: (0, i, 0)),
            pl.BlockSpec((_NC, _BLK, _D_EDGE), lambda i: (0, i, 0)),
            pl.BlockSpec((_D_IN + _D_EDGE, _D_OUT), lambda i: (0, 0)),
            pl.BlockSpec((1, _D_OUT), lambda i: (0, 0)),
        ],
        out_specs=pl.BlockSpec((_BLK, _D_OUT), lambda i: (i, 0)),
        out_shape=jax.ShapeDtypeStruct((_N_NODES, _D_OUT), jnp.float32),
    )(px, pe, w_msg, bias.reshape(1, _D_OUT))


def kernel(x, edge_index, edge_attr, W_msg, bias):
    src = edge_index[0].astype(jnp.int32)
    dst = edge_index[1].astype(jnp.int32)
    px, pe = _sc_segment_sums(x, src, dst, edge_attr)
    return _tc_combine(px, pe, W_msg, bias)


# speedup vs baseline: 2.6161x; 2.6161x over previous
import functools

import jax
import jax.numpy as jnp
from jax import lax
from jax.experimental import pallas as pl
from jax.experimental.pallas import tpu as pltpu
from jax.experimental.pallas import tpu_sc as plsc

_N_NODES = 10000
_N_EDGES = 320000
_D_IN = 128
_D_EDGE = 16
_D_OUT = 128

_NC = 2
_NS = 16
_NW = _NC * _NS

_K = 80
_EDGES_PER_WORKER = _N_EDGES // _NW
_CHUNKS_PER_WORKER = _EDGES_PER_WORKER // _K
_N_PAD = 10240
_ROWS_PER_SUB = _N_PAD // _NS


def _sc_segsum(v, src, dst, z, zeros_x):
    mesh = plsc.VectorSubcoreMesh(core_axis_name="c", subcore_axis_name="s")

    @functools.partial(
        pl.kernel,
        out_type=jax.ShapeDtypeStruct((_NC, _N_PAD, _D_OUT), jnp.float32),
        mesh=mesh,
        scratch_types=[
            pltpu.VMEM_SHARED((_N_PAD, _D_OUT), jnp.float32),
            pltpu.VMEM((_K,), jnp.int32),
            pltpu.VMEM((_K,), jnp.int32),
            pltpu.VMEM((_K, _D_OUT), jnp.float32),
            pltpu.VMEM((_K, _D_OUT), jnp.float32),
        ],
    )
    def k(v_hbm, src_hbm, dst_hbm, z_hbm, zx_hbm, out_hbm,
          acc, src_v, dst_v, rows_v, zrows_v):
        c = lax.axis_index("c")
        s = lax.axis_index("s")
        w = c * _NS + s
        r0 = s * _ROWS_PER_SUB

        pltpu.sync_copy(zx_hbm, acc.at[pl.ds(r0, _ROWS_PER_SUB)])
        plsc.subcore_barrier()

        base = w * _EDGES_PER_WORKER

        @pl.loop(0, _CHUNKS_PER_WORKER)
        def _(i):
            off = base + i * _K
            pltpu.sync_copy(src_hbm.at[pl.ds(off, _K)], src_v)
            pltpu.sync_copy(dst_hbm.at[pl.ds(off, _K)], dst_v)
            pltpu.sync_copy(v_hbm.at[src_v], rows_v)
            pltpu.sync_copy(rows_v, acc.at[dst_v], add=True)
            pltpu.sync_copy(z_hbm.at[pl.ds(off, _K)], zrows_v)
            pltpu.sync_copy(zrows_v, acc.at[dst_v], add=True)

        plsc.subcore_barrier()
        pltpu.sync_copy(acc.at[pl.ds(r0, _ROWS_PER_SUB)],
                        out_hbm.at[c, pl.ds(r0, _ROWS_PER_SUB)])

    return k(v, src, dst, z, zeros_x)


def _tc_matmul(a, w, blk):
    m, kd = a.shape

    def body(a_ref, w_ref, o_ref):
        o_ref[...] = lax.dot_general(
            a_ref[...], w_ref[...], (((1,), (0,)), ((), ())),
            precision=lax.Precision.HIGHEST,
            preferred_element_type=jnp.float32)

    return pl.pallas_call(
        body,
        grid=(m // blk,),
        in_specs=[pl.BlockSpec((blk, kd), lambda i: (i, 0)),
                  pl.BlockSpec((kd, _D_OUT), lambda i: (0, 0))],
        out_specs=pl.BlockSpec((blk, _D_OUT), lambda i: (i, 0)),
        out_shape=jax.ShapeDtypeStruct((m, _D_OUT), jnp.float32),
    )(a, w)


_BLK = 1000


def _tc_combine(px, bias):
    def body(px_ref, b_ref, o_ref):
        o_ref[...] = px_ref[0] + px_ref[1] + b_ref[...]

    return pl.pallas_call(
        body,
        grid=(_N_NODES // _BLK,),
        in_specs=[pl.BlockSpec((_NC, _BLK, _D_OUT), lambda i: (0, i, 0)),
                  pl.BlockSpec((1, _D_OUT), lambda i: (0, 0))],
        out_specs=pl.BlockSpec((_BLK, _D_OUT), lambda i: (i, 0)),
        out_shape=jax.ShapeDtypeStruct((_N_NODES, _D_OUT), jnp.float32),
    )(px, bias.reshape(1, _D_OUT))


def kernel(x, edge_index, edge_attr, W_msg, bias):
    src = edge_index[0].astype(jnp.int32)
    dst = edge_index[1].astype(jnp.int32)
    v = _tc_matmul(x, W_msg[:_D_IN], 1000)
    z = _tc_matmul(edge_attr, W_msg[_D_IN:], 4000)
    zeros_x = jnp.zeros((_ROWS_PER_SUB, _D_OUT), jnp.float32)
    px = _sc_segsum(v, src, dst, z, zeros_x)
    return _tc_combine(px, bias)


# speedup vs baseline: 3.2295x; 1.2345x over previous
import functools

import jax
import jax.numpy as jnp
from jax import lax
from jax.experimental import pallas as pl
from jax.experimental.pallas import tpu as pltpu
from jax.experimental.pallas import tpu_sc as plsc

_N_NODES = 10000
_N_EDGES = 320000
_D_IN = 128
_D_EDGE = 16
_D_OUT = 128

_NC = 2
_NS = 16
_NW = _NC * _NS

_K = 40
_EDGES_PER_WORKER = _N_EDGES // _NW
_CHUNKS = _EDGES_PER_WORKER // _K
_PAIRS = _CHUNKS // 2
_N_PAD = 10240
_ROWS_PER_SUB = _N_PAD // _NS


def _sc_segsum(v, src, dst, z, zeros_x):
    mesh = plsc.VectorSubcoreMesh(core_axis_name="c", subcore_axis_name="s")

    @functools.partial(
        pl.kernel,
        out_type=jax.ShapeDtypeStruct((_NC, _N_PAD, _D_OUT), jnp.float32),
        mesh=mesh,
        scratch_types=[
            pltpu.VMEM_SHARED((_N_PAD, _D_OUT), jnp.float32),
            pltpu.VMEM((_K,), jnp.int32),
            pltpu.VMEM((_K,), jnp.int32),
            pltpu.VMEM((_K,), jnp.int32),
            pltpu.VMEM((_K,), jnp.int32),
            pltpu.VMEM((_K, _D_OUT), jnp.float32),
            pltpu.VMEM((_K, _D_OUT), jnp.float32),
            pltpu.VMEM((_K, _D_OUT), jnp.float32),
            pltpu.VMEM((_K, _D_OUT), jnp.float32),
            pltpu.SemaphoreType.DMA,
            pltpu.SemaphoreType.DMA,
            pltpu.SemaphoreType.DMA,
            pltpu.SemaphoreType.DMA,
            pltpu.SemaphoreType.DMA,
            pltpu.SemaphoreType.DMA,
            pltpu.SemaphoreType.DMA,
            pltpu.SemaphoreType.DMA,
        ],
    )
    def k(v_hbm, src_hbm, dst_hbm, z_hbm, zx_hbm, out_hbm, acc,
          s0, s1, d0, d1, rb0, rb1, zb0, zb1,
          gsem0, gsem1, zsem0, zsem1, rsem0, rsem1, qsem0, qsem1):
        S = (s0, s1)
        D = (d0, d1)
        RB = (rb0, rb1)
        ZB = (zb0, zb1)
        GSEM = (gsem0, gsem1)
        ZSEM = (zsem0, zsem1)
        RSEM = (rsem0, rsem1)
        QSEM = (qsem0, qsem1)

        c = lax.axis_index("c")
        sc = lax.axis_index("s")
        w = c * _NS + sc
        r0 = sc * _ROWS_PER_SUB

        pltpu.sync_copy(zx_hbm, acc.at[pl.ds(r0, _ROWS_PER_SUB)])
        plsc.subcore_barrier()

        base = w * _EDGES_PER_WORKER

        def load_and_start(b, j):
            off = base + j * _K
            pltpu.sync_copy(src_hbm.at[pl.ds(off, _K)], S[b])
            pltpu.sync_copy(dst_hbm.at[pl.ds(off, _K)], D[b])
            pltpu.async_copy(v_hbm.at[S[b]], RB[b], GSEM[b])
            pltpu.async_copy(z_hbm.at[pl.ds(off, _K)], ZB[b], ZSEM[b])

        for b in (0, 1):
            load_and_start(b, b)

        @pl.loop(0, _PAIRS)
        def _(i):
            for b in (0, 1):
                j = 2 * i + b
                pltpu.make_async_copy(v_hbm.at[S[b]], RB[b], GSEM[b]).wait()
                pltpu.async_copy(RB[b], acc.at[D[b]], RSEM[b], add=True)
                pltpu.make_async_copy(z_hbm.at[pl.ds(base, _K)], ZB[b],
                                      ZSEM[b]).wait()
                pltpu.async_copy(ZB[b], acc.at[D[b]], QSEM[b], add=True)
                pltpu.make_async_copy(RB[b], acc.at[D[b]], RSEM[b]).wait()
                pltpu.make_async_copy(ZB[b], acc.at[D[b]], QSEM[b]).wait()

                @pl.when(j + 2 < _CHUNKS)
                def _():
                    load_and_start(b, j + 2)

        plsc.subcore_barrier()
        pltpu.sync_copy(acc.at[pl.ds(r0, _ROWS_PER_SUB)],
                        out_hbm.at[c, pl.ds(r0, _ROWS_PER_SUB)])

    return k(v, src, dst, z, zeros_x)


def _tc_matmul(a, w, blk):
    m, kd = a.shape

    def body(a_ref, w_ref, o_ref):
        o_ref[...] = lax.dot_general(
            a_ref[...], w_ref[...], (((1,), (0,)), ((), ())),
            precision=lax.Precision.HIGHEST,
            preferred_element_type=jnp.float32)

    return pl.pallas_call(
        body,
        grid=(m // blk,),
        in_specs=[pl.BlockSpec((blk, kd), lambda i: (i, 0)),
                  pl.BlockSpec((kd, _D_OUT), lambda i: (0, 0))],
        out_specs=pl.BlockSpec((blk, _D_OUT), lambda i: (i, 0)),
        out_shape=jax.ShapeDtypeStruct((m, _D_OUT), jnp.float32),
    )(a, w)


_BLK = 1000


def _tc_combine(px, bias):
    def body(px_ref, b_ref, o_ref):
        o_ref[...] = px_ref[0] + px_ref[1] + b_ref[...]

    return pl.pallas_call(
        body,
        grid=(_N_NODES // _BLK,),
        in_specs=[pl.BlockSpec((_NC, _BLK, _D_OUT), lambda i: (0, i, 0)),
                  pl.BlockSpec((1, _D_OUT), lambda i: (0, 0))],
        out_specs=pl.BlockSpec((_BLK, _D_OUT), lambda i: (i, 0)),
        out_shape=jax.ShapeDtypeStruct((_N_NODES, _D_OUT), jnp.float32),
    )(px, bias.reshape(1, _D_OUT))


def kernel(x, edge_index, edge_attr, W_msg, bias):
    src = edge_index[0].astype(jnp.int32)
    dst = edge_index[1].astype(jnp.int32)
    v = _tc_matmul(x, W_msg[:_D_IN], 1000)
    z = _tc_matmul(edge_attr, W_msg[_D_IN:], 4000)
    zeros_x = jnp.zeros((_ROWS_PER_SUB, _D_OUT), jnp.float32)
    px = _sc_segsum(v, src, dst, z, zeros_x)
    return _tc_combine(px, bias)


# speedup vs baseline: 3.4880x; 1.0800x over previous
import functools

import jax
import jax.numpy as jnp
from jax import lax
from jax.experimental import pallas as pl
from jax.experimental.pallas import tpu as pltpu
from jax.experimental.pallas import tpu_sc as plsc

_N_NODES = 10000
_N_EDGES = 320000
_D_IN = 128
_D_EDGE = 16
_D_OUT = 128

_NC = 2
_NS = 16
_NW = _NC * _NS

_K = 40
_EDGES_PER_WORKER = _N_EDGES // _NW
_CHUNKS = _EDGES_PER_WORKER // _K
_PAIRS = _CHUNKS // 2
_N_PAD = 10240
_ROWS_PER_SUB = _N_PAD // _NS


def _sc_segsum(v, src, dst, z, zeros_x):
    mesh = plsc.VectorSubcoreMesh(core_axis_name="c", subcore_axis_name="s")

    @functools.partial(
        pl.kernel,
        out_type=jax.ShapeDtypeStruct((_NC, _N_PAD, _D_OUT), jnp.float32),
        mesh=mesh,
        scratch_types=[
            pltpu.VMEM_SHARED((_N_PAD, _D_OUT), jnp.float32),
            pltpu.VMEM((_K,), jnp.int32),
            pltpu.VMEM((_K,), jnp.int32),
            pltpu.VMEM((_K,), jnp.int32),
            pltpu.VMEM((_K,), jnp.int32),
            pltpu.VMEM((_K, _D_OUT), jnp.float32),
            pltpu.VMEM((_K, _D_OUT), jnp.float32),
            pltpu.VMEM((_K, _D_OUT), jnp.float32),
            pltpu.VMEM((_K, _D_OUT), jnp.float32),
            pltpu.SemaphoreType.DMA,
            pltpu.SemaphoreType.DMA,
            pltpu.SemaphoreType.DMA,
            pltpu.SemaphoreType.DMA,
            pltpu.SemaphoreType.DMA,
            pltpu.SemaphoreType.DMA,
            pltpu.SemaphoreType.DMA,
            pltpu.SemaphoreType.DMA,
        ],
    )
    def k(v_hbm, src_hbm, dst_hbm, z_hbm, zx_hbm, out_hbm, acc,
          s0, s1, d0, d1, rb0, rb1, zb0, zb1,
          gsem0, gsem1, zsem0, zsem1, rsem0, rsem1, qsem0, qsem1):
        S = (s0, s1)
        D = (d0, d1)
        RB = (rb0, rb1)
        ZB = (zb0, zb1)
        GSEM = (gsem0, gsem1)
        ZSEM = (zsem0, zsem1)
        RSEM = (rsem0, rsem1)
        QSEM = (qsem0, qsem1)

        c = lax.axis_index("c")
        sc = lax.axis_index("s")
        w = c * _NS + sc
        r0 = sc * _ROWS_PER_SUB

        pltpu.sync_copy(zx_hbm, acc.at[pl.ds(r0, _ROWS_PER_SUB)])
        plsc.subcore_barrier()

        base = w * _EDGES_PER_WORKER

        def load_and_start(b, j):
            off = base + j * _K
            pltpu.sync_copy(src_hbm.at[pl.ds(off, _K)], S[b])
            pltpu.sync_copy(dst_hbm.at[pl.ds(off, _K)], D[b])
            pltpu.async_copy(v_hbm.at[S[b]], RB[b], GSEM[b])
            pltpu.async_copy(z_hbm.at[pl.ds(off, _K)], ZB[b], ZSEM[b])

        for b in (0, 1):
            load_and_start(b, b)

        @pl.loop(0, _PAIRS)
        def _(i):
            for b in (0, 1):
                j = 2 * i + b
                pltpu.make_async_copy(v_hbm.at[S[b]], RB[b], GSEM[b]).wait()
                pltpu.async_copy(RB[b], acc.at[D[b]], RSEM[b], add=True)
                pltpu.make_async_copy(z_hbm.at[pl.ds(base, _K)], ZB[b],
                                      ZSEM[b]).wait()
                pltpu.async_copy(ZB[b], acc.at[D[b]], QSEM[b], add=True)
                pltpu.make_async_copy(RB[b], acc.at[D[b]], RSEM[b]).wait()
                pltpu.make_async_copy(ZB[b], acc.at[D[b]], QSEM[b]).wait()

                @pl.when(j + 2 < _CHUNKS)
                def _():
                    load_and_start(b, j + 2)

        plsc.subcore_barrier()
        pltpu.sync_copy(acc.at[pl.ds(r0, _ROWS_PER_SUB)],
                        out_hbm.at[c, pl.ds(r0, _ROWS_PER_SUB)])

    return k(v, src, dst, z, zeros_x)


def _tc_matmul(a, w, blk, bf16=False):
    m, kd = a.shape

    def body(a_ref, w_ref, o_ref):
        av, wv = a_ref[...], w_ref[...]
        if bf16:
            av = av.astype(jnp.bfloat16)
            wv = wv.astype(jnp.bfloat16)
        o_ref[...] = lax.dot_general(
            av, wv, (((1,), (0,)), ((), ())),
            precision=None if bf16 else lax.Precision.HIGHEST,
            preferred_element_type=jnp.float32)

    return pl.pallas_call(
        body,
        grid=(m // blk,),
        in_specs=[pl.BlockSpec((blk, kd), lambda i: (i, 0)),
                  pl.BlockSpec((kd, _D_OUT), lambda i: (0, 0))],
        out_specs=pl.BlockSpec((blk, _D_OUT), lambda i: (i, 0)),
        out_shape=jax.ShapeDtypeStruct((m, _D_OUT), jnp.float32),
    )(a, w)


_BLK = 1000


def _tc_combine(px, bias):
    def body(px_ref, b_ref, o_ref):
        o_ref[...] = px_ref[0] + px_ref[1] + b_ref[...]

    return pl.pallas_call(
        body,
        grid=(_N_NODES // _BLK,),
        in_specs=[pl.BlockSpec((_NC, _BLK, _D_OUT), lambda i: (0, i, 0)),
                  pl.BlockSpec((1, _D_OUT), lambda i: (0, 0))],
        out_specs=pl.BlockSpec((_BLK, _D_OUT), lambda i: (i, 0)),
        out_shape=jax.ShapeDtypeStruct((_N_NODES, _D_OUT), jnp.float32),
    )(px, bias.reshape(1, _D_OUT))


def kernel(x, edge_index, edge_attr, W_msg, bias):
    src = edge_index[0].astype(jnp.int32)
    dst = edge_index[1].astype(jnp.int32)
    v = _tc_matmul(x, W_msg[:_D_IN], 1000)
    z = _tc_matmul(edge_attr, W_msg[_D_IN:], 4000, bf16=True)
    zeros_x = jnp.zeros((_ROWS_PER_SUB, _D_OUT), jnp.float32)
    px = _sc_segsum(v, src, dst, z, zeros_x)
    return _tc_combine(px, bias)


# speedup vs baseline: 4.3535x; 1.2481x over previous
import functools

import jax
import jax.numpy as jnp
from jax import lax
from jax.experimental import pallas as pl
from jax.experimental.pallas import tpu as pltpu
from jax.experimental.pallas import tpu_sc as plsc

_N_NODES = 10000
_N_EDGES = 320000
_D_IN = 128
_D_EDGE = 16
_D_OUT = 128

_NC = 2
_NS = 16
_NW = _NC * _NS

_K = 80
_EDGES_PER_WORKER = _N_EDGES // _NW
_CHUNKS = _EDGES_PER_WORKER // _K
_PAIRS = _CHUNKS // 2
_N_PAD = 10240
_ROWS_PER_SUB = _N_PAD // _NS


def _sc_segsum(v, src, dst, z, zeros_x):
    mesh = plsc.VectorSubcoreMesh(core_axis_name="c", subcore_axis_name="s")

    @functools.partial(
        pl.kernel,
        out_type=jax.ShapeDtypeStruct((_NC, _N_PAD, _D_OUT), jnp.float32),
        mesh=mesh,
        scratch_types=[
            pltpu.VMEM_SHARED((_N_PAD, _D_OUT), jnp.float32),
            pltpu.VMEM((_K,), jnp.int32),
            pltpu.VMEM((_K,), jnp.int32),
            pltpu.VMEM((_K,), jnp.int32),
            pltpu.VMEM((_K,), jnp.int32),
            pltpu.VMEM((_K, _D_OUT), jnp.float32),
            pltpu.VMEM((_K, _D_OUT), jnp.float32),
            pltpu.VMEM((_K, _D_OUT), jnp.float32),
            pltpu.SemaphoreType.DMA,
            pltpu.SemaphoreType.DMA,
            pltpu.SemaphoreType.DMA,
            pltpu.SemaphoreType.DMA,
            pltpu.SemaphoreType.DMA,
            pltpu.SemaphoreType.DMA,
        ],
    )
    def k(v_hbm, src_hbm, dst_hbm, z_hbm, zx_hbm, out_hbm, acc,
          s0, s1, d0, d1, rb0, rb1, zb,
          gsem0, gsem1, zsem, rsem0, rsem1, qsem):
        S = (s0, s1)
        D = (d0, d1)
        RB = (rb0, rb1)
        GSEM = (gsem0, gsem1)
        RSEM = (rsem0, rsem1)

        c = lax.axis_index("c")
        sc = lax.axis_index("s")
        w = c * _NS + sc
        r0 = sc * _ROWS_PER_SUB

        pltpu.sync_copy(zx_hbm, acc.at[pl.ds(r0, _ROWS_PER_SUB)])
        plsc.subcore_barrier()

        base = w * _EDGES_PER_WORKER

        def load_and_gather(b, j):
            off = base + j * _K
            pltpu.sync_copy(src_hbm.at[pl.ds(off, _K)], S[b])
            pltpu.sync_copy(dst_hbm.at[pl.ds(off, _K)], D[b])
            pltpu.async_copy(v_hbm.at[S[b]], RB[b], GSEM[b])

        def zload(j):
            off = base + j * _K
            pltpu.async_copy(z_hbm.at[pl.ds(off, _K)], zb, zsem)

        for b in (0, 1):
            load_and_gather(b, b)
        zload(0)

        def block(i, b):
            j = 2 * i + b
            pltpu.make_async_copy(v_hbm.at[S[b]], RB[b], GSEM[b]).wait()
            pltpu.async_copy(RB[b], acc.at[D[b]], RSEM[b], add=True)
            pltpu.make_async_copy(z_hbm.at[pl.ds(base, _K)], zb, zsem).wait()
            pltpu.async_copy(zb, acc.at[D[b]], qsem, add=True)
            pltpu.make_async_copy(zb, acc.at[D[b]], qsem).wait()

            @pl.when(j + 1 < _CHUNKS)
            def _():
                zload(j + 1)

            pltpu.make_async_copy(RB[b], acc.at[D[b]], RSEM[b]).wait()

            @pl.when(j + 2 < _CHUNKS)
            def _():
                load_and_gather(b, j + 2)

        @pl.loop(0, _PAIRS)
        def _(i):
            for b in (0, 1):
                block(i, b)

        block(_PAIRS, 0)

        plsc.subcore_barrier()
        pltpu.sync_copy(acc.at[pl.ds(r0, _ROWS_PER_SUB)],
                        out_hbm.at[c, pl.ds(r0, _ROWS_PER_SUB)])

    return k(v, src, dst, z, zeros_x)


def _tc_matmul(a, w, blk, bf16=False):
    m, kd = a.shape

    def body(a_ref, w_ref, o_ref):
        av, wv = a_ref[...], w_ref[...]
        if bf16:
            av = av.astype(jnp.bfloat16)
            wv = wv.astype(jnp.bfloat16)
        o_ref[...] = lax.dot_general(
            av, wv, (((1,), (0,)), ((), ())),
            precision=None if bf16 else lax.Precision.HIGHEST,
            preferred_element_type=jnp.float32)

    return pl.pallas_call(
        body,
        grid=(m // blk,),
        in_specs=[pl.BlockSpec((blk, kd), lambda i: (i, 0)),
                  pl.BlockSpec((kd, _D_OUT), lambda i: (0, 0))],
        out_specs=pl.BlockSpec((blk, _D_OUT), lambda i: (i, 0)),
        out_shape=jax.ShapeDtypeStruct((m, _D_OUT), jnp.float32),
    )(a, w)


_BLK = 1000


def _tc_combine(px, bias):
    def body(px_ref, b_ref, o_ref):
        o_ref[...] = px_ref[0] + px_ref[1] + b_ref[...]

    return pl.pallas_call(
        body,
        grid=(_N_NODES // _BLK,),
        in_specs=[pl.BlockSpec((_NC, _BLK, _D_OUT), lambda i: (0, i, 0)),
                  pl.BlockSpec((1, _D_OUT), lambda i: (0, 0))],
        out_specs=pl.BlockSpec((_BLK, _D_OUT), lambda i: (i, 0)),
        out_shape=jax.ShapeDtypeStruct((_N_NODES, _D_OUT), jnp.float32),
    )(px, bias.reshape(1, _D_OUT))


def kernel(x, edge_index, edge_attr, W_msg, bias):
    src = edge_index[0].astype(jnp.int32)
    dst = edge_index[1].astype(jnp.int32)
    v = _tc_matmul(x, W_msg[:_D_IN], 1000)
    z = _tc_matmul(edge_attr, W_msg[_D_IN:], 8000, bf16=True)
    zeros_x = jnp.zeros((_ROWS_PER_SUB, _D_OUT), jnp.float32)
    px = _sc_segsum(v, src, dst, z, zeros_x)
    return _tc_combine(px, bias)


# speedup vs baseline: 4.4623x; 1.0250x over previous
import functools

import jax
import jax.numpy as jnp
from jax import lax
from jax.experimental import pallas as pl
from jax.experimental.pallas import tpu as pltpu
from jax.experimental.pallas import tpu_sc as plsc

_N_NODES = 10000
_N_EDGES = 320000
_D_IN = 128
_D_EDGE = 16
_D_OUT = 128

_NC = 2
_NS = 16
_NW = _NC * _NS

_K = 80
_EDGES_PER_WORKER = _N_EDGES // _NW
_CHUNKS = _EDGES_PER_WORKER // _K
_PAIRS = _CHUNKS // 2
_N_PAD = 10240
_ROWS_PER_SUB = _N_PAD // _NS

_MESH = plsc.VectorSubcoreMesh(core_axis_name="c", subcore_axis_name="s")


def _sc_gather_segsum(v, src, dst, zeros_x):
    @functools.partial(
        pl.kernel,
        out_type=jax.ShapeDtypeStruct((_NC, _N_PAD, _D_OUT), jnp.float32),
        mesh=_MESH,
        scratch_types=[
            pltpu.VMEM_SHARED((_N_PAD, _D_OUT), jnp.float32),
            pltpu.VMEM((_K,), jnp.int32),
            pltpu.VMEM((_K,), jnp.int32),
            pltpu.VMEM((_K,), jnp.int32),
            pltpu.VMEM((_K,), jnp.int32),
            pltpu.VMEM((_K, _D_OUT), jnp.float32),
            pltpu.VMEM((_K, _D_OUT), jnp.float32),
            pltpu.SemaphoreType.DMA,
            pltpu.SemaphoreType.DMA,
            pltpu.SemaphoreType.DMA,
            pltpu.SemaphoreType.DMA,
        ],
    )
    def k(v_hbm, src_hbm, dst_hbm, zx_hbm, out_hbm, acc,
          s0, s1, d0, d1, rb0, rb1, gsem0, gsem1, rsem0, rsem1):
        S = (s0, s1)
        D = (d0, d1)
        RB = (rb0, rb1)
        GSEM = (gsem0, gsem1)
        RSEM = (rsem0, rsem1)

        c = lax.axis_index("c")
        sc = lax.axis_index("s")
        w = c * _NS + sc
        r0 = sc * _ROWS_PER_SUB

        pltpu.sync_copy(zx_hbm, acc.at[pl.ds(r0, _ROWS_PER_SUB)])
        plsc.subcore_barrier()

        base = w * _EDGES_PER_WORKER

        def load_and_gather(b, j):
            off = base + j * _K
            pltpu.sync_copy(src_hbm.at[pl.ds(off, _K)], S[b])
            pltpu.sync_copy(dst_hbm.at[pl.ds(off, _K)], D[b])
            pltpu.async_copy(v_hbm.at[S[b]], RB[b], GSEM[b])

        for b in (0, 1):
            load_and_gather(b, b)

        def block(i, b):
            j = 2 * i + b
            pltpu.make_async_copy(v_hbm.at[S[b]], RB[b], GSEM[b]).wait()
            pltpu.async_copy(RB[b], acc.at[D[b]], RSEM[b], add=True)
            pltpu.make_async_copy(RB[b], acc.at[D[b]], RSEM[b]).wait()

            @pl.when(j + 2 < _CHUNKS)
            def _():
                load_and_gather(b, j + 2)

        @pl.loop(0, _PAIRS)
        def _(i):
            for b in (0, 1):
                block(i, b)

        block(_PAIRS, 0)

        plsc.subcore_barrier()
        pltpu.sync_copy(acc.at[pl.ds(r0, _ROWS_PER_SUB)],
                        out_hbm.at[c, pl.ds(r0, _ROWS_PER_SUB)])

    return k(v, src, dst, zeros_x)


def _sc_linear_segsum(z, dst, zeros_x):
    @functools.partial(
        pl.kernel,
        out_type=jax.ShapeDtypeStruct((_NC, _N_PAD, _D_OUT), jnp.float32),
        mesh=_MESH,
        scratch_types=[
            pltpu.VMEM_SHARED((_N_PAD, _D_OUT), jnp.float32),
            pltpu.VMEM((_K,), jnp.int32),
            pltpu.VMEM((_K,), jnp.int32),
            pltpu.VMEM((_K, _D_OUT), jnp.float32),
            pltpu.VMEM((_K, _D_OUT), jnp.float32),
            pltpu.SemaphoreType.DMA,
            pltpu.SemaphoreType.DMA,
            pltpu.SemaphoreType.DMA,
            pltpu.SemaphoreType.DMA,
        ],
    )
    def k(z_hbm, dst_hbm, zx_hbm, out_hbm, acc,
          d0, d1, zb0, zb1, zsem0, zsem1, qsem0, qsem1):
        D = (d0, d1)
        ZB = (zb0, zb1)
        ZSEM = (zsem0, zsem1)
        QSEM = (qsem0, qsem1)

        c = lax.axis_index("c")
        sc = lax.axis_index("s")
        w = c * _NS + sc
        r0 = sc * _ROWS_PER_SUB

        pltpu.sync_copy(zx_hbm, acc.at[pl.ds(r0, _ROWS_PER_SUB)])
        plsc.subcore_barrier()

        base = w * _EDGES_PER_WORKER

        def load_chunk(b, j):
            off = base + j * _K
            pltpu.sync_copy(dst_hbm.at[pl.ds(off, _K)], D[b])
            pltpu.async_copy(z_hbm.at[pl.ds(off, _K)], ZB[b], ZSEM[b])

        for b in (0, 1):
            load_chunk(b, b)

        def block(i, b):
            j = 2 * i + b
            pltpu.make_async_copy(z_hbm.at[pl.ds(base, _K)], ZB[b],
                                  ZSEM[b]).wait()
            pltpu.async_copy(ZB[b], acc.at[D[b]], QSEM[b], add=True)
            pltpu.make_async_copy(ZB[b], acc.at[D[b]], QSEM[b]).wait()

            @pl.when(j + 2 < _CHUNKS)
            def _():
                load_chunk(b, j + 2)

        @pl.loop(0, _PAIRS)
        def _(i):
            for b in (0, 1):
                block(i, b)

        block(_PAIRS, 0)

        plsc.subcore_barrier()
        pltpu.sync_copy(acc.at[pl.ds(r0, _ROWS_PER_SUB)],
                        out_hbm.at[c, pl.ds(r0, _ROWS_PER_SUB)])

    return k(z, dst, zeros_x)


def _tc_matmul(a, w, blk, bf16=False):
    m, kd = a.shape

    def body(a_ref, w_ref, o_ref):
        av, wv = a_ref[...], w_ref[...]
        if bf16:
            av = av.astype(jnp.bfloat16)
            wv = wv.astype(jnp.bfloat16)
        o_ref[...] = lax.dot_general(
            av, wv, (((1,), (0,)), ((), ())),
            precision=None if bf16 else lax.Precision.HIGHEST,
            preferred_element_type=jnp.float32)

    return pl.pallas_call(
        body,
        grid=(m // blk,),
        in_specs=[pl.BlockSpec((blk, kd), lambda i: (i, 0)),
                  pl.BlockSpec((kd, _D_OUT), lambda i: (0, 0))],
        out_specs=pl.BlockSpec((blk, _D_OUT), lambda i: (i, 0)),
        out_shape=jax.ShapeDtypeStruct((m, _D_OUT), jnp.float32),
    )(a, w)


_BLK = 1000


def _tc_combine(pv, pz, bias):
    def body(pv_ref, pz_ref, b_ref, o_ref):
        o_ref[...] = (pv_ref[0] + pv_ref[1]) + (pz_ref[0] + pz_ref[1]) + b_ref[...]

    return pl.pallas_call(
        body,
        grid=(_N_NODES // _BLK,),
        in_specs=[pl.BlockSpec((_NC, _BLK, _D_OUT), lambda i: (0, i, 0)),
                  pl.BlockSpec((_NC, _BLK, _D_OUT), lambda i: (0, i, 0)),
                  pl.BlockSpec((1, _D_OUT), lambda i: (0, 0))],
        out_specs=pl.BlockSpec((_BLK, _D_OUT), lambda i: (i, 0)),
        out_shape=jax.ShapeDtypeStruct((_N_NODES, _D_OUT), jnp.float32),
    )(pv, pz, bias.reshape(1, _D_OUT))


def kernel(x, edge_index, edge_attr, W_msg, bias):
    src = edge_index[0].astype(jnp.int32)
    dst = edge_index[1].astype(jnp.int32)
    zeros_x = jnp.zeros((_ROWS_PER_SUB, _D_OUT), jnp.float32)
    v = _tc_matmul(x, W_msg[:_D_IN], 1000)
    pv = _sc_gather_segsum(v, src, dst, zeros_x)
    z = _tc_matmul(edge_attr, W_msg[_D_IN:], 8000, bf16=True)
    pz = _sc_linear_segsum(z, dst, zeros_x)
    return _tc_combine(pv, pz, bias)
